# Initial kernel scaffold; baseline (speedup 1.0000x reference)
import jax
import jax.numpy as jnp
from jax.experimental import pallas as pl


def kernel(feature):
    raise NotImplementedError("write your pallas kernel here")


# speedup vs baseline: 4.2905x; 4.2905x over previous
import functools

import jax
import jax.numpy as jnp
from jax.experimental import pallas as pl
import jax.experimental.pallas.tpu as pltpu

N = 8192
D = 512
K = 32
R = 256
NUM_BLOCKS = N // R


def _normalize_body(x_ref, o_ref):
    x = x_ref[...]
    nrm = jnp.sqrt(jnp.sum(x * x, axis=1, keepdims=True))
    o_ref[...] = x / (nrm + 1e-10)


def _topk_body(a_ref, b_ref, vals_ref, idx_ref, sim_ref, amprev_ref):
    i = pl.program_id(0)
    j = pl.program_id(1)

    col = jax.lax.broadcasted_iota(jnp.int32, (R, N), 1)

    @pl.when(j == 0)
    def _():
        a = a_ref[...]
        b = b_ref[...]
        sim = jax.lax.dot_general(
            a, b, (((1,), (1,)), ((), ())),
            preferred_element_type=jnp.float32)
        row = jax.lax.broadcasted_iota(jnp.int32, (R, N), 0) + i * R
        sim_ref[...] = jnp.where(col == row, 0.0, sim)

    s = sim_ref[...]

    @pl.when(j > 0)
    def _():
        am_prev = amprev_ref[0, :]
        sim_ref[...] = jnp.where(col == am_prev[:, None], -jnp.inf, s)

    s = sim_ref[...]
    m = jnp.max(s, axis=1)
    am = jnp.min(jnp.where(s == m[:, None], col, N), axis=1).astype(jnp.int32)
    vals_ref[0, 0, :] = m
    idx_ref[0, 0, :] = am
    amprev_ref[0, :] = am


@functools.partial(jax.jit)
def kernel(feature):
    nf = pl.pallas_call(
        _normalize_body,
        grid=(8,),
        in_specs=[pl.BlockSpec((N // 8, D), lambda i: (i, 0))],
        out_specs=pl.BlockSpec((N // 8, D), lambda i: (i, 0)),
        out_shape=jax.ShapeDtypeStruct((N, D), jnp.float32),
    )(feature)

    vals_t, idx_t = pl.pallas_call(
        _topk_body,
        grid=(NUM_BLOCKS, K),
        in_specs=[
            pl.BlockSpec((R, D), lambda i, j: (i, 0)),
            pl.BlockSpec((N, D), lambda i, j: (0, 0)),
        ],
        out_specs=[
            pl.BlockSpec((1, 1, R), lambda i, j: (j, 0, i)),
            pl.BlockSpec((1, 1, R), lambda i, j: (j, 0, i)),
        ],
        out_shape=[
            jax.ShapeDtypeStruct((K, 1, N), jnp.float32),
            jax.ShapeDtypeStruct((K, 1, N), jnp.int32),
        ],
        scratch_shapes=[
            pltpu.VMEM((R, N), jnp.float32),
            pltpu.VMEM((1, R), jnp.int32),
        ],
        compiler_params=pltpu.CompilerParams(
            dimension_semantics=("arbitrary", "arbitrary"),
        ),
    )(nf, nf)

    src = jnp.tile(jnp.arange(N, dtype=jnp.int32), K)
    edge = jnp.stack([src, idx_t.reshape(-1)])
    edge_weights = vals_t.reshape(-1)
    return (edge, edge_weights)


# speedup vs baseline: 4.3844x; 1.0219x over previous
import functools

import jax
import jax.numpy as jnp
from jax.experimental import pallas as pl
import jax.experimental.pallas.tpu as pltpu

N = 8192
D = 512
K = 32
R = 256
C = 128
NC = N // C
NUM_BLOCKS = N // R
NCACHE = 4

NEG = float("-inf")
BIG = N


def _normalize_body(x_ref, o_ref):
    x = x_ref[...]
    nrm = jnp.sqrt(jnp.sum(x * x, axis=1, keepdims=True))
    o_ref[...] = x / (nrm + 1e-10)


def _topk_body(a_ref, b_ref, vals_ref, idx_ref, s3_ref, vc_ref, cc_ref):
    i = pl.program_id(0)
    j = pl.program_id(1)

    @pl.when(j == 0)
    def _():
        a = a_ref[...]
        b = b_ref[...]
        sim = jax.lax.dot_general(
            a, b, (((1,), (1,)), ((), ())),
            preferred_element_type=jnp.float32)
        col = jax.lax.broadcasted_iota(jnp.int32, (R, N), 1)
        row = jax.lax.broadcasted_iota(jnp.int32, (R, N), 0) + i * R
        sim = jnp.where(col == row, 0.0, sim)
        s3 = sim.reshape(R, NC, C)
        s3_ref[...] = s3
        lane3 = jax.lax.broadcasted_iota(jnp.int32, (R, NC, C), 2)
        giota2 = jax.lax.broadcasted_iota(jnp.int32, (R, NC), 1)
        t = jnp.max(s3, axis=2)
        l = jnp.min(jnp.where(s3 == t[:, :, None], lane3, C), axis=2)
        vc_ref[0] = t
        cc_ref[0] = giota2 * C + l
        for k in range(1, NCACHE):
            keep = (s3 < t[:, :, None]) | (
                (s3 == t[:, :, None]) & (lane3 > l[:, :, None]))
            s3m = jnp.where(keep, s3, NEG)
            t = jnp.max(s3m, axis=2)
            l = jnp.min(jnp.where(s3m == t[:, :, None], lane3, C), axis=2)
            vc_ref[k] = t
            cc_ref[k] = giota2 * C + l

    giota2 = jax.lax.broadcasted_iota(jnp.int32, (R, NC), 1)
    v0 = vc_ref[0]
    m = jnp.max(v0, axis=1)
    gstar = jnp.min(jnp.where(v0 == m[:, None], giota2, NC), axis=1)
    onehot = giota2 == gstar[:, None]
    am = jnp.max(jnp.where(onehot, cc_ref[0], -1), axis=1)

    for k in range(NCACHE - 1):
        vc_ref[k] = jnp.where(onehot, vc_ref[k + 1], vc_ref[k])
        cc_ref[k] = jnp.where(onehot, cc_ref[k + 1], cc_ref[k])
    vc_ref[NCACHE - 1] = jnp.where(onehot, NEG, vc_ref[NCACHE - 1])

    head = jnp.max(jnp.where(onehot, vc_ref[0], NEG), axis=1)
    need = head == NEG

    @pl.when(jnp.any(need))
    def _():
        bias2 = jnp.where(need[:, None] & onehot, 0.0, NEG)
        sext = jnp.max(s3_ref[...] + bias2[:, :, None], axis=1)
        lane = jax.lax.broadcasted_iota(jnp.int32, (R, C), 1)
        colg = gstar[:, None] * C + lane
        mb = m[:, None]
        rem = jnp.where((sext < mb) | ((sext == mb) & (colg > am[:, None])),
                        sext, NEG)
        wmask = need[:, None] & onehot
        for k in range(NCACHE):
            t = jnp.max(rem, axis=1)
            c = jnp.min(jnp.where(rem == t[:, None], colg, BIG), axis=1)
            vc_ref[k] = jnp.where(wmask, t[:, None], vc_ref[k])
            cc_ref[k] = jnp.where(wmask, c[:, None], cc_ref[k])
            rem = jnp.where(colg == c[:, None], NEG, rem)

    vals_ref[0, 0, :] = m
    idx_ref[0, 0, :] = am


@functools.partial(jax.jit)
def kernel(feature):
    nf = pl.pallas_call(
        _normalize_body,
        grid=(8,),
        in_specs=[pl.BlockSpec((N // 8, D), lambda i: (i, 0))],
        out_specs=pl.BlockSpec((N // 8, D), lambda i: (i, 0)),
        out_shape=jax.ShapeDtypeStruct((N, D), jnp.float32),
    )(feature)

    vals_t, idx_t = pl.pallas_call(
        _topk_body,
        grid=(NUM_BLOCKS, K),
        in_specs=[
            pl.BlockSpec((R, D), lambda i, j: (i, 0)),
            pl.BlockSpec((N, D), lambda i, j: (0, 0)),
        ],
        out_specs=[
            pl.BlockSpec((1, 1, R), lambda i, j: (j, 0, i)),
            pl.BlockSpec((1, 1, R), lambda i, j: (j, 0, i)),
        ],
        out_shape=[
            jax.ShapeDtypeStruct((K, 1, N), jnp.float32),
            jax.ShapeDtypeStruct((K, 1, N), jnp.int32),
        ],
        scratch_shapes=[
            pltpu.VMEM((R, NC, C), jnp.float32),
            pltpu.VMEM((NCACHE, R, NC), jnp.float32),
            pltpu.VMEM((NCACHE, R, NC), jnp.int32),
        ],
        compiler_params=pltpu.CompilerParams(
            dimension_semantics=("arbitrary", "arbitrary"),
        ),
    )(nf, nf)

    src = jnp.tile(jnp.arange(N, dtype=jnp.int32), K)
    edge = jnp.stack([src, idx_t.reshape(-1)])
    edge_weights = vals_t.reshape(-1)
    return (edge, edge_weights)


# speedup vs baseline: 4.3880x; 1.0008x over previous
import functools

import jax
import jax.numpy as jnp
from jax.experimental import pallas as pl
import jax.experimental.pallas.tpu as pltpu

N = 8192
D = 512
K = 32
R = 256
C = 128
NC = N // C
NUM_BLOCKS = N // R
NCACHE = 4

NEG = float("-inf")
BIG = N


def _normalize_body(x_ref, o_ref):
    x = x_ref[...]
    nrm = jnp.sqrt(jnp.sum(x * x, axis=1, keepdims=True))
    o_ref[...] = x / (nrm + 1e-10)


def _topk_body(a_ref, b_ref, vals_ref, idx_ref, s3_ref, vc_ref, cc_ref,
               b_vmem_ref, sem):
    i = pl.program_id(0)
    j = pl.program_id(1)

    @pl.when((i == 0) & (j == 0))
    def _():
        pltpu.make_async_copy(b_ref, b_vmem_ref, sem).start()
        pltpu.make_async_copy(b_ref, b_vmem_ref, sem).wait()

    @pl.when(j == 0)
    def _():
        a = a_ref[...]
        b = b_vmem_ref[...]
        sim = jax.lax.dot_general(
            a, b, (((1,), (1,)), ((), ())),
            preferred_element_type=jnp.float32)
        col = jax.lax.broadcasted_iota(jnp.int32, (R, N), 1)
        row = jax.lax.broadcasted_iota(jnp.int32, (R, N), 0) + i * R
        sim = jnp.where(col == row, 0.0, sim)
        s3 = sim.reshape(R, NC, C)
        s3_ref[...] = s3
        lane3 = jax.lax.broadcasted_iota(jnp.int32, (R, NC, C), 2)
        giota2 = jax.lax.broadcasted_iota(jnp.int32, (R, NC), 1)
        t = jnp.max(s3, axis=2)
        l = jnp.min(jnp.where(s3 == t[:, :, None], lane3, C), axis=2)
        vc_ref[0] = t
        cc_ref[0] = giota2 * C + l
        for k in range(1, NCACHE):
            keep = (s3 < t[:, :, None]) | (
                (s3 == t[:, :, None]) & (lane3 > l[:, :, None]))
            s3m = jnp.where(keep, s3, NEG)
            t = jnp.max(s3m, axis=2)
            l = jnp.min(jnp.where(s3m == t[:, :, None], lane3, C), axis=2)
            vc_ref[k] = t
            cc_ref[k] = giota2 * C + l

    giota2 = jax.lax.broadcasted_iota(jnp.int32, (R, NC), 1)
    v0 = vc_ref[0]
    m = jnp.max(v0, axis=1)
    gstar = jnp.min(jnp.where(v0 == m[:, None], giota2, NC), axis=1)
    onehot = giota2 == gstar[:, None]
    am = jnp.max(jnp.where(onehot, cc_ref[0], -1), axis=1)

    for k in range(NCACHE - 1):
        vc_ref[k] = jnp.where(onehot, vc_ref[k + 1], vc_ref[k])
        cc_ref[k] = jnp.where(onehot, cc_ref[k + 1], cc_ref[k])
    vc_ref[NCACHE - 1] = jnp.where(onehot, NEG, vc_ref[NCACHE - 1])

    head = jnp.max(jnp.where(onehot, vc_ref[0], NEG), axis=1)
    need = head == NEG

    @pl.when(jnp.any(need))
    def _():
        bias2 = jnp.where(need[:, None] & onehot, 0.0, NEG)
        sext = jnp.max(s3_ref[...] + bias2[:, :, None], axis=1)
        lane = jax.lax.broadcasted_iota(jnp.int32, (R, C), 1)
        colg = gstar[:, None] * C + lane
        mb = m[:, None]
        rem = jnp.where((sext < mb) | ((sext == mb) & (colg > am[:, None])),
                        sext, NEG)
        wmask = need[:, None] & onehot
        for k in range(NCACHE):
            t = jnp.max(rem, axis=1)
            c = jnp.min(jnp.where(rem == t[:, None], colg, BIG), axis=1)
            vc_ref[k] = jnp.where(wmask, t[:, None], vc_ref[k])
            cc_ref[k] = jnp.where(wmask, c[:, None], cc_ref[k])
            rem = jnp.where(colg == c[:, None], NEG, rem)

    vals_ref[0, 0, :] = m
    idx_ref[0, 0, :] = am


@functools.partial(jax.jit)
def kernel(feature):
    nf = pl.pallas_call(
        _normalize_body,
        grid=(8,),
        in_specs=[pl.BlockSpec((N // 8, D), lambda i: (i, 0))],
        out_specs=pl.BlockSpec((N // 8, D), lambda i: (i, 0)),
        out_shape=jax.ShapeDtypeStruct((N, D), jnp.float32),
    )(feature)

    vals_t, idx_t = pl.pallas_call(
        _topk_body,
        grid=(NUM_BLOCKS, K),
        in_specs=[
            pl.BlockSpec((R, D), lambda i, j: (i, 0)),
            pl.BlockSpec(memory_space=pl.ANY),
        ],
        out_specs=[
            pl.BlockSpec((1, 1, R), lambda i, j: (j, 0, i)),
            pl.BlockSpec((1, 1, R), lambda i, j: (j, 0, i)),
        ],
        out_shape=[
            jax.ShapeDtypeStruct((K, 1, N), jnp.float32),
            jax.ShapeDtypeStruct((K, 1, N), jnp.int32),
        ],
        scratch_shapes=[
            pltpu.VMEM((R, NC, C), jnp.float32),
            pltpu.VMEM((NCACHE, R, NC), jnp.float32),
            pltpu.VMEM((NCACHE, R, NC), jnp.int32),
            pltpu.VMEM((N, D), jnp.float32),
            pltpu.SemaphoreType.DMA,
        ],
        compiler_params=pltpu.CompilerParams(
            dimension_semantics=("arbitrary", "arbitrary"),
        ),
    )(nf, nf)

    src = jnp.tile(jnp.arange(N, dtype=jnp.int32), K)
    edge = jnp.stack([src, idx_t.reshape(-1)])
    edge_weights = vals_t.reshape(-1)
    return (edge, edge_weights)
